# Initial kernel scaffold; baseline (speedup 1.0000x reference)
import jax
import jax.numpy as jnp
from jax.experimental import pallas as pl


def kernel(x, token_to_slot, weight, lora_a, lora_b_q, lora_b_k, lora_b_v, lora_scaling):
    raise NotImplementedError("write your pallas kernel here")


# speedup vs baseline: 4.4786x; 4.4786x over previous
import functools

import jax
import jax.numpy as jnp
from jax import lax
from jax.experimental import pallas as pl
from jax.experimental.pallas import tpu as pltpu
from jax.experimental.pallas import tpu_sc as plsc

H = 2048
L = 8
R = 16
T = 4096
OUT = 6144
NTGT = 3
AC = NTGT * L * R
GRP = L * R


_NC, _NS, _LANES = 2, 16, 16
_NW = _NC * _NS
_TPW = T // _NW


def _mask_body(slots_hbm, rows_hbm, m_hbm, idx_v, m_v, sem):
    wid = lax.axis_index("s") * _NC + lax.axis_index("c")
    base = wid * _TPW
    pltpu.sync_copy(slots_hbm.at[pl.ds(base, _TPW)], idx_v)
    pltpu.async_copy(rows_hbm.at[idx_v], m_v, sem).wait()
    pltpu.sync_copy(m_v, m_hbm.at[pl.ds(base, _TPW)])


def _build_mask(token_to_slot, rows):
    mesh = plsc.VectorSubcoreMesh(core_axis_name="c", subcore_axis_name="s")
    kern = functools.partial(
        pl.kernel,
        mesh=mesh,
        out_type=jax.ShapeDtypeStruct((T, AC), jnp.float32),
        scratch_types=[
            pltpu.VMEM((_TPW,), jnp.int32),
            pltpu.VMEM((_TPW, AC), jnp.float32),
            pltpu.SemaphoreType.DMA,
        ],
    )(_mask_body)
    return kern(token_to_slot, rows)


_BT1 = 512


def _ax_body(x_ref, a_ref, o_ref):
    o_ref[...] = lax.dot_general(
        x_ref[...], a_ref[...], (((1,), (1,)), ((), ())),
        preferred_element_type=jnp.float32)


def _compute_ax(x, a_stack):
    return pl.pallas_call(
        _ax_body,
        grid=(T // _BT1,),
        in_specs=[
            pl.BlockSpec((_BT1, H), lambda i: (i, 0)),
            pl.BlockSpec((AC, H), lambda i: (0, 0)),
        ],
        out_specs=pl.BlockSpec((_BT1, AC), lambda i: (i, 0)),
        out_shape=jax.ShapeDtypeStruct((T, AC), jnp.float32),
        compiler_params=pltpu.CompilerParams(
            dimension_semantics=("arbitrary",)),
    )(x, a_stack)


_BT = 512
_BN = 1024
_JPT = (OUT // NTGT) // _BN


def _main_body(x_ref, w_ref, ax_ref, m_ref, b_ref, o_ref):
    base = lax.dot_general(
        x_ref[...], w_ref[...], (((1,), (1,)), ((), ())),
        preferred_element_type=jnp.float32)
    axm = ax_ref[...] * m_ref[...]
    delta = lax.dot_general(
        axm, b_ref[0], (((1,), (0,)), ((), ())),
        preferred_element_type=jnp.float32)
    o_ref[...] = base + delta


def _main_matmul(x, weight, ax, m, b_flat):
    return pl.pallas_call(
        _main_body,
        grid=(OUT // _BN, T // _BT),
        in_specs=[
            pl.BlockSpec((_BT, H), lambda j, i: (i, 0)),
            pl.BlockSpec((_BN, H), lambda j, i: (j, 0)),
            pl.BlockSpec((_BT, GRP), lambda j, i: (i, j // _JPT)),
            pl.BlockSpec((_BT, GRP), lambda j, i: (i, j // _JPT)),
            pl.BlockSpec((1, GRP, _BN), lambda j, i: (j // _JPT, 0, j % _JPT)),
        ],
        out_specs=pl.BlockSpec((_BT, _BN), lambda j, i: (i, j)),
        out_shape=jax.ShapeDtypeStruct((T, OUT), jnp.float32),
        compiler_params=pltpu.CompilerParams(
            dimension_semantics=("arbitrary", "arbitrary")),
    )(x, weight, ax, m, b_flat)


def kernel(x, token_to_slot, weight, lora_a, lora_b_q, lora_b_k, lora_b_v,
           lora_scaling):
    a_stack = lora_a.transpose(1, 0, 2, 3).reshape(AC, H)
    b_flat = jnp.stack([
        lora_b_q.transpose(0, 2, 1).reshape(GRP, OUT // NTGT),
        lora_b_k.transpose(0, 2, 1).reshape(GRP, OUT // NTGT),
        lora_b_v.transpose(0, 2, 1).reshape(GRP, OUT // NTGT),
    ])
    rows = jnp.tile(
        jnp.repeat(lora_scaling[:, None] * jnp.eye(L, dtype=jnp.float32),
                   R, axis=1),
        (1, NTGT))

    m = _build_mask(token_to_slot, rows)
    ax = _compute_ax(x, a_stack)
    return _main_matmul(x, weight, ax, m, b_flat)


# speedup vs baseline: 4.9365x; 1.1022x over previous
import functools

import jax
import jax.numpy as jnp
from jax import lax
from jax.experimental import pallas as pl
from jax.experimental.pallas import tpu as pltpu
from jax.experimental.pallas import tpu_sc as plsc

H = 2048
L = 8
R = 16
T = 4096
OUT = 6144
NTGT = 3
AC = NTGT * L * R
GRP = L * R


_NC, _NS, _LANES = 2, 16, 16
_NW = _NC * _NS
_TPW = T // _NW


def _mask_body(slots_hbm, rows_hbm, m_hbm, idx_v, m_v, sem):
    wid = lax.axis_index("s") * _NC + lax.axis_index("c")
    base = wid * _TPW
    pltpu.sync_copy(slots_hbm.at[pl.ds(base, _TPW)], idx_v)
    pltpu.async_copy(rows_hbm.at[idx_v], m_v, sem).wait()
    pltpu.sync_copy(m_v, m_hbm.at[pl.ds(base, _TPW)])


def _build_mask(token_to_slot, rows):
    mesh = plsc.VectorSubcoreMesh(core_axis_name="c", subcore_axis_name="s")
    kern = functools.partial(
        pl.kernel,
        mesh=mesh,
        out_type=jax.ShapeDtypeStruct((T, GRP), jnp.float32),
        scratch_types=[
            pltpu.VMEM((_TPW,), jnp.int32),
            pltpu.VMEM((_TPW, GRP), jnp.float32),
            pltpu.SemaphoreType.DMA,
        ],
    )(_mask_body)
    return kern(token_to_slot, rows)


_BT1 = 512


def _ax_body(x_ref, a_ref, o_ref):
    o_ref[...] = lax.dot_general(
        x_ref[...], a_ref[...], (((1,), (1,)), ((), ())),
        preferred_element_type=jnp.float32)


def _compute_ax(x, a_stack):
    return pl.pallas_call(
        _ax_body,
        grid=(T // _BT1,),
        in_specs=[
            pl.BlockSpec((_BT1, H), lambda i: (i, 0)),
            pl.BlockSpec((AC, H), lambda i: (0, 0)),
        ],
        out_specs=pl.BlockSpec((_BT1, AC), lambda i: (i, 0)),
        out_shape=jax.ShapeDtypeStruct((T, AC), jnp.float32),
        compiler_params=pltpu.CompilerParams(
            dimension_semantics=("parallel",)),
    )(x, a_stack)


_BT = 512
_BN = 2048


def _main_body(x_ref, w_ref, ax_ref, m_ref, b_ref, o_ref):
    base = lax.dot_general(
        x_ref[...], w_ref[...], (((1,), (1,)), ((), ())),
        preferred_element_type=jnp.float32)
    axm = ax_ref[...] * m_ref[...]
    delta = lax.dot_general(
        axm, b_ref[0], (((1,), (0,)), ((), ())),
        preferred_element_type=jnp.float32)
    o_ref[...] = base + delta


def _main_matmul(x, weight, ax, m, b_flat):
    return pl.pallas_call(
        _main_body,
        grid=(OUT // _BN, T // _BT),
        in_specs=[
            pl.BlockSpec((_BT, H), lambda j, i: (i, 0)),
            pl.BlockSpec((_BN, H), lambda j, i: (j, 0)),
            pl.BlockSpec((_BT, GRP), lambda j, i: (i, j)),
            pl.BlockSpec((_BT, GRP), lambda j, i: (i, 0)),
            pl.BlockSpec((1, GRP, _BN), lambda j, i: (j, 0, 0)),
        ],
        out_specs=pl.BlockSpec((_BT, _BN), lambda j, i: (i, j)),
        out_shape=jax.ShapeDtypeStruct((T, OUT), jnp.float32),
        compiler_params=pltpu.CompilerParams(
            dimension_semantics=("parallel", "parallel")),
    )(x, weight, ax, m, b_flat)


def kernel(x, token_to_slot, weight, lora_a, lora_b_q, lora_b_k, lora_b_v,
           lora_scaling):
    a_stack = lora_a.transpose(1, 0, 2, 3).reshape(AC, H)
    b_flat = jnp.stack([
        lora_b_q.transpose(0, 2, 1).reshape(GRP, OUT // NTGT),
        lora_b_k.transpose(0, 2, 1).reshape(GRP, OUT // NTGT),
        lora_b_v.transpose(0, 2, 1).reshape(GRP, OUT // NTGT),
    ])
    rows = jnp.repeat(lora_scaling[:, None] * jnp.eye(L, dtype=jnp.float32),
                      R, axis=1)

    m = _build_mask(token_to_slot, rows)
    ax = _compute_ax(x, a_stack)
    return _main_matmul(x, weight, ax, m, b_flat)
